# baseline (speedup 1.0000x reference)
import functools

import jax
import jax.numpy as jnp
from jax import lax
from jax.experimental import pallas as pl
from jax.experimental.pallas import tpu as pltpu
from jax.experimental.pallas import tpu_sc as plsc

N = 10000
E = 320000
D = 128
NUM_RBF = 32
CUTOFF = 6.0
STEP = CUTOFF / (NUM_RBF - 1)
GAMMA = 1.0 / max(STEP * STEP, 1e-06)

BN = 1000
BE = 2000

_NC = 2
_NS = 16
_NW = _NC * _NS
_L = 16

_NH = 2
_EH = E // _NH
_EPW = _EH // _NW

_GCH = 200
_GSUBS = ((0, 80), (80, 80), (160, 40))
_GNCH = _EPW // _GCH

_SCH = 200
_SSUBS = ((0, 104), (104, 96))
_SNCH = _EPW // _SCH
_RPT = 624
_RTAIL = N - _NS * _RPT


def _silu(x):
    return x * (1.0 / (1.0 + jnp.exp(-x)))


def _table_body(h_ref, w1at_ref, w1bt_ref, a_ref, b_ref):
    h = h_ref[...]
    a_ref[...] = jnp.dot(h, w1at_ref[...], preferred_element_type=jnp.float32)
    b_ref[...] = jnp.dot(h, w1bt_ref[...], preferred_element_type=jnp.float32)


def _build_tables(h, w1at, w1bt):
    grid = N // BN
    return pl.pallas_call(
        _table_body,
        grid=(grid,),
        in_specs=[
            pl.BlockSpec((BN, D), lambda i: (i, 0)),
            pl.BlockSpec((D, D), lambda i: (0, 0)),
            pl.BlockSpec((D, D), lambda i: (0, 0)),
        ],
        out_specs=[
            pl.BlockSpec((BN, D), lambda i: (i, 0)),
            pl.BlockSpec((BN, D), lambda i: (i, 0)),
        ],
        out_shape=[
            jax.ShapeDtypeStruct((N, D), jnp.float32),
            jax.ShapeDtypeStruct((N, D), jnp.float32),
        ],
    )(h, w1at, w1bt)


def _sc_mesh():
    return plsc.VectorSubcoreMesh(core_axis_name="c", subcore_axis_name="s")


def _sc_gather_body(ta, tb, srcr, dstr, pxr, pyr, pzr,
                    ga, gb, d2o, cnto,
                    idxs, idxd, buf, d2b, cntb, px, py, pz, sem):
    cid = lax.axis_index("c")
    sid = lax.axis_index("s")
    wid = sid * _NC + cid
    base = wid * _EPW

    pltpu.sync_copy(pxr, px)
    pltpu.sync_copy(pyr, py)
    pltpu.sync_copy(pzr, pz)

    zero16 = jnp.zeros((_L,), jnp.float32)

    def zinit(r, carry):
        cntb[pl.ds(r * _L, _L)] = zero16
        return carry

    lax.fori_loop(0, N // _L, zinit, 0)

    one16 = jnp.ones((_L,), jnp.float32)

    def chunk(i, carry):
        off = base + i * _GCH
        pltpu.sync_copy(srcr.at[pl.ds(off, _GCH)], idxs)
        pltpu.sync_copy(dstr.at[pl.ds(off, _GCH)], idxd)
        ha = []
        for (o, sz) in _GSUBS:
            sl = pl.ds(o, sz)
            ha.append(pltpu.async_copy(ta.at[idxs.at[sl]], buf.at[sl], sem))

        def dcomp(k, c2):
            s16 = pl.ds(k * _L, _L)
            iv_s = idxs[s16]
            iv_d = idxd[s16]
            dx = plsc.load_gather(px, [iv_s]) - plsc.load_gather(px, [iv_d])
            dy = plsc.load_gather(py, [iv_s]) - plsc.load_gather(py, [iv_d])
            dz = plsc.load_gather(pz, [iv_s]) - plsc.load_gather(pz, [iv_d])
            d2b[s16] = dx * dx + dy * dy + dz * dz
            plsc.addupdate_scatter(cntb, [iv_d], one16)
            return c2

        lax.fori_loop(0, _GCH // _L, dcomp, 0)
        pltpu.sync_copy(d2b, d2o.at[pl.ds(off, _GCH)])

        for h in ha:
            h.wait()
        pltpu.sync_copy(buf, ga.at[pl.ds(off, _GCH)])
        hb = []
        for (o, sz) in _GSUBS:
            sl = pl.ds(o, sz)
            hb.append(pltpu.async_copy(tb.at[idxd.at[sl]], buf.at[sl], sem))
        for h in hb:
            h.wait()
        pltpu.sync_copy(buf, gb.at[pl.ds(off, _GCH)])
        return carry

    lax.fori_loop(0, _GNCH, chunk, 0)
    for blk in range(N // BN):
        pltpu.sync_copy(cntb.at[pl.ds(blk * BN, BN)],
                        cnto.at[pl.ds((blk * _NW + wid) * BN, BN)])


def _sc_gather(ta, tb, src, dst, px, py, pz):
    f = pl.kernel(
        _sc_gather_body,
        out_type=[
            jax.ShapeDtypeStruct((_EH, D), jnp.float32),
            jax.ShapeDtypeStruct((_EH, D), jnp.float32),
            jax.ShapeDtypeStruct((_EH,), jnp.float32),
            jax.ShapeDtypeStruct((N * _NW,), jnp.float32),
        ],
        mesh=_sc_mesh(),
        scratch_types=[
            pltpu.VMEM((_GCH,), jnp.int32),
            pltpu.VMEM((_GCH,), jnp.int32),
            pltpu.VMEM((_GCH, D), jnp.float32),
            pltpu.VMEM((_GCH,), jnp.float32),
            pltpu.VMEM((N,), jnp.float32),
            pltpu.VMEM((N,), jnp.float32),
            pltpu.VMEM((N,), jnp.float32),
            pltpu.VMEM((N,), jnp.float32),
            pltpu.SemaphoreType.DMA,
        ],
        compiler_params=pltpu.CompilerParams(needs_layout_passes=False),
    )
    return f(ta, tb, src, dst, px, py, pz)


def _sc_scatter_body(msgp, dstr, zer, out, shared, msgbuf, idxv0, idxv1, sem):
    cid = lax.axis_index("c")
    sid = lax.axis_index("s")
    wid = sid * _NC + cid
    rows = pl.ds(sid * _RPT, _RPT)
    tail = pl.ds(_NS * _RPT, _RTAIL)
    pltpu.sync_copy(zer.at[cid, rows], shared.at[rows])

    @pl.when(sid == 0)
    def _():
        pltpu.sync_copy(zer.at[cid, tail], shared.at[tail])

    plsc.subcore_barrier()
    base = wid * _EPW

    def chunk(i, carry):
        off = base + i * _SCH
        pltpu.sync_copy(msgp.at[pl.ds(off, _SCH)], msgbuf)
        (o0, z0), (o1, z1) = _SSUBS
        pltpu.sync_copy(dstr.at[pl.ds(off + o0, z0)], idxv0)
        pltpu.sync_copy(dstr.at[pl.ds(off + o1, z1)], idxv1)
        h0 = pltpu.async_copy(msgbuf.at[pl.ds(o0, z0)],
                              shared.at[idxv0], sem, add=True)
        h1 = pltpu.async_copy(msgbuf.at[pl.ds(o1, z1)],
                              shared.at[idxv1], sem, add=True)
        h0.wait()
        h1.wait()
        return carry

    lax.fori_loop(0, _SNCH, chunk, 0)
    plsc.subcore_barrier()
    pltpu.sync_copy(shared.at[rows], out.at[cid, rows])

    @pl.when(sid == 0)
    def _():
        pltpu.sync_copy(shared.at[tail], out.at[cid, tail])


def _sc_scatter(msgp, dstr, init):
    f = pl.kernel(
        _sc_scatter_body,
        out_type=jax.ShapeDtypeStruct((_NC, N, D), jnp.float32),
        mesh=_sc_mesh(),
        scratch_types=[
            pltpu.MemorySpace.VMEM_SHARED((N, D), jnp.float32),
            pltpu.VMEM((_SCH, D), jnp.float32),
            pltpu.VMEM((_SSUBS[0][1],), jnp.int32),
            pltpu.VMEM((_SSUBS[1][1],), jnp.int32),
            pltpu.SemaphoreType.DMA,
        ],
        compiler_params=pltpu.CompilerParams(needs_layout_passes=False),
    )
    return f(msgp, dstr, init)


def _edge_body(ga_ref, gb_ref, d2_ref, et_ref, emb_ref, w1ct_ref, w1rt_ref,
               w1d_ref, b1_ref, w2t_ref, b2_ref, out_ref):
    ga = ga_ref[...]
    gb = gb_ref[...]
    dist = jnp.sqrt(d2_ref[...])
    centers = STEP * lax.broadcasted_iota(jnp.int32, (1, NUM_RBF), 1).astype(jnp.float32)
    diff = dist - centers
    radial = jnp.exp(-GAMMA * diff * diff)
    t = jnp.dot(emb_ref[...], w1ct_ref[...], preferred_element_type=jnp.float32) \
        + b1_ref[...]
    et = et_ref[...]
    tsel = t[0:1, :] * (1.0 - et) + t[1:2, :] * et
    pre1 = (ga + gb + tsel
            + jnp.dot(radial, w1rt_ref[...], preferred_element_type=jnp.float32)
            + dist * w1d_ref[...])
    x = _silu(pre1)
    out_ref[...] = _silu(
        jnp.dot(x, w2t_ref[...], preferred_element_type=jnp.float32)
        + b2_ref[...])


def _edge_mlp(ga, gb, d2, etf, emb, w1ct, w1rt, w1d, b1, w2t, b2):
    grid = _EH // BE
    full = lambda i: (0, 0)
    return pl.pallas_call(
        _edge_body,
        grid=(grid,),
        in_specs=[
            pl.BlockSpec((BE, D), lambda i: (i, 0)),
            pl.BlockSpec((BE, D), lambda i: (i, 0)),
            pl.BlockSpec((BE, 1), lambda i: (i, 0)),
            pl.BlockSpec((BE, 1), lambda i: (i, 0)),
            pl.BlockSpec((2, D), full),
            pl.BlockSpec((D, D), full),
            pl.BlockSpec((NUM_RBF, D), full),
            pl.BlockSpec((1, D), full),
            pl.BlockSpec((1, D), full),
            pl.BlockSpec((D, D), full),
            pl.BlockSpec((1, D), full),
        ],
        out_specs=pl.BlockSpec((BE, D), lambda i: (i, 0)),
        out_shape=jax.ShapeDtypeStruct((_EH, D), jnp.float32),
    )(ga, gb, d2, etf, emb, w1ct, w1rt, w1d, b1, w2t, b2)


def _node_body(h_ref, s0_ref, s1_ref, cnt0_ref, cnt1_ref, nt_ref, u1at_ref,
               u1bt_ref, c1_ref, u2t_ref, c2_ref, g_ref, bta_ref, out_ref):
    h = h_ref[...]
    s = s0_ref[...] + s1_ref[...]
    cnt = jnp.sum(cnt0_ref[0] + cnt1_ref[0], axis=0, keepdims=True)
    recip = 1.0 / jnp.maximum(cnt, 1.0)
    ii = lax.broadcasted_iota(jnp.int32, (BN, BN), 0)
    jj = lax.broadcasted_iota(jnp.int32, (BN, BN), 1)
    dg = jnp.where(ii == jj, recip, 0.0)
    agg = jnp.dot(dg, s, preferred_element_type=jnp.float32)
    u = _silu(jnp.dot(h, u1at_ref[...], preferred_element_type=jnp.float32)
              + jnp.dot(agg, u1bt_ref[...], preferred_element_type=jnp.float32)
              + c1_ref[...])
    upd = jnp.dot(u, u2t_ref[...], preferred_element_type=jnp.float32) + c2_ref[...]
    pre = h + upd
    mu = jnp.mean(pre, axis=1, keepdims=True)
    cent = pre - mu
    var = jnp.mean(cent * cent, axis=1, keepdims=True)
    ln = cent * lax.rsqrt(var + 1e-05) * g_ref[...] + bta_ref[...]
    out_ref[...] = jnp.where(nt_ref[...] == 1.0, ln, h)


def _node_update(h, s0, s1, cnt0, cnt1, ntf, u1at, u1bt, c1, u2t, c2, g, b):
    grid = N // BN
    full = lambda i: (0, 0)
    return pl.pallas_call(
        _node_body,
        grid=(grid,),
        in_specs=[
            pl.BlockSpec((BN, D), lambda i: (i, 0)),
            pl.BlockSpec((BN, D), lambda i: (i, 0)),
            pl.BlockSpec((BN, D), lambda i: (i, 0)),
            pl.BlockSpec((1, _NW, BN), lambda i: (i, 0, 0)),
            pl.BlockSpec((1, _NW, BN), lambda i: (i, 0, 0)),
            pl.BlockSpec((BN, 1), lambda i: (i, 0)),
            pl.BlockSpec((D, D), full),
            pl.BlockSpec((D, D), full),
            pl.BlockSpec((1, D), full),
            pl.BlockSpec((D, D), full),
            pl.BlockSpec((1, D), full),
            pl.BlockSpec((1, D), full),
            pl.BlockSpec((1, D), full),
        ],
        out_specs=pl.BlockSpec((BN, D), lambda i: (i, 0)),
        out_shape=jax.ShapeDtypeStruct((N, D), jnp.float32),
    )(h, s0, s1, cnt0, cnt1, ntf, u1at, u1bt, c1, u2t, c2, g, b)


def kernel(h, pos, edge_index, edge_type, node_type, emb, W1, b1, W2, b2,
           U1, c1, U2, c2, gamma_ln, beta_ln):
    src32 = edge_index[0].astype(jnp.int32)
    dst32 = edge_index[1].astype(jnp.int32)
    w1at = W1[:, :D].T
    w1bt = W1[:, D:2 * D].T
    w1ct = W1[:, 2 * D:3 * D].T
    w1rt = W1[:, 3 * D:3 * D + NUM_RBF].T
    w1d = W1[:, 3 * D + NUM_RBF][None, :]
    b1r = b1[None, :]
    w2t = W2.T
    b2r = b2[None, :]
    u1at = U1[:, :D].T
    u1bt = U1[:, D:].T
    c1r = c1[None, :]
    u2t = U2.T
    c2r = c2[None, :]
    gr = gamma_ln[None, :]
    br = beta_ln[None, :]
    px = pos[:, 0]
    py = pos[:, 1]
    pz = pos[:, 2]

    ta, tb = _build_tables(h, w1at, w1bt)

    etf = edge_type.astype(jnp.float32)[:, None]

    cnts = []
    parts = jnp.zeros((_NC, N, D), jnp.float32)
    for half in range(_NH):
        esl = slice(half * _EH, (half + 1) * _EH)
        dhalf = dst32[esl]
        ga, gb, d2, cntf = _sc_gather(ta, tb, src32[esl], dhalf, px, py, pz)
        cnts.append(cntf.reshape(N // BN, _NW, BN))
        msgp = _edge_mlp(ga, gb, d2[:, None], etf[esl], emb, w1ct, w1rt,
                         w1d, b1r, w2t, b2r)
        parts = _sc_scatter(msgp, dhalf, parts)

    ntf = node_type.astype(jnp.float32)[:, None]
    return _node_update(h, parts[0], parts[1], cnts[0], cnts[1], ntf, u1at,
                        u1bt, c1r, u2t, c2r, gr, br)


# speedup vs baseline: 1.0296x; 1.0296x over previous
import functools

import jax
import jax.numpy as jnp
from jax import lax
from jax.experimental import pallas as pl
from jax.experimental.pallas import tpu as pltpu
from jax.experimental.pallas import tpu_sc as plsc

N = 10000
E = 320000
D = 128
NUM_RBF = 32
CUTOFF = 6.0
STEP = CUTOFF / (NUM_RBF - 1)
GAMMA = 1.0 / max(STEP * STEP, 1e-06)

BN = 1000
BE = 2000

_NC = 2
_NS = 16
_NW = _NC * _NS
_L = 16

_NH = 1
_EH = E // _NH
_EPW = _EH // _NW

_GCH = 200
_GSUBS = ((0, 80), (80, 80), (160, 40))
_GNCH = _EPW // _GCH

_SCH = 200
_SSUBS = ((0, 104), (104, 96))
_SNCH = _EPW // _SCH
_RPT = 624
_RTAIL = N - _NS * _RPT


def _silu(x):
    return x * (1.0 / (1.0 + jnp.exp(-x)))


def _table_body(h_ref, w1at_ref, w1bt_ref, a_ref, b_ref):
    h = h_ref[...]
    a_ref[...] = jnp.dot(h, w1at_ref[...], preferred_element_type=jnp.float32)
    b_ref[...] = jnp.dot(h, w1bt_ref[...], preferred_element_type=jnp.float32)


def _build_tables(h, w1at, w1bt):
    grid = N // BN
    return pl.pallas_call(
        _table_body,
        grid=(grid,),
        in_specs=[
            pl.BlockSpec((BN, D), lambda i: (i, 0)),
            pl.BlockSpec((D, D), lambda i: (0, 0)),
            pl.BlockSpec((D, D), lambda i: (0, 0)),
        ],
        out_specs=[
            pl.BlockSpec((BN, D), lambda i: (i, 0)),
            pl.BlockSpec((BN, D), lambda i: (i, 0)),
        ],
        out_shape=[
            jax.ShapeDtypeStruct((N, D), jnp.float32),
            jax.ShapeDtypeStruct((N, D), jnp.float32),
        ],
    )(h, w1at, w1bt)


def _sc_mesh():
    return plsc.VectorSubcoreMesh(core_axis_name="c", subcore_axis_name="s")


def _sc_gather_body(ta, tb, srcr, dstr, pxr, pyr, pzr,
                    gs, d2o, cnto,
                    idxs, idxd, bufa, bufb, d2b, cntb, px, py, pz, sem):
    cid = lax.axis_index("c")
    sid = lax.axis_index("s")
    wid = sid * _NC + cid
    base = wid * _EPW

    pltpu.sync_copy(pxr, px)
    pltpu.sync_copy(pyr, py)
    pltpu.sync_copy(pzr, pz)

    zero16 = jnp.zeros((_L,), jnp.float32)

    def zinit(r, carry):
        cntb[pl.ds(r * _L, _L)] = zero16
        return carry

    lax.fori_loop(0, N // _L, zinit, 0)

    one16 = jnp.ones((_L,), jnp.float32)

    def chunk(i, carry):
        off = base + i * _GCH
        pltpu.sync_copy(srcr.at[pl.ds(off, _GCH)], idxs)
        pltpu.sync_copy(dstr.at[pl.ds(off, _GCH)], idxd)
        hs = []
        for (o, sz) in _GSUBS:
            sl = pl.ds(o, sz)
            hs.append(pltpu.async_copy(ta.at[idxs.at[sl]], bufa.at[sl], sem))
            hs.append(pltpu.async_copy(tb.at[idxd.at[sl]], bufb.at[sl], sem))

        def dcomp(k, c2):
            s16 = pl.ds(k * _L, _L)
            iv_s = idxs[s16]
            iv_d = idxd[s16]
            dx = plsc.load_gather(px, [iv_s]) - plsc.load_gather(px, [iv_d])
            dy = plsc.load_gather(py, [iv_s]) - plsc.load_gather(py, [iv_d])
            dz = plsc.load_gather(pz, [iv_s]) - plsc.load_gather(pz, [iv_d])
            d2b[s16] = dx * dx + dy * dy + dz * dz
            plsc.addupdate_scatter(cntb, [iv_d], one16)
            return c2

        lax.fori_loop(0, _GCH // _L, dcomp, 0)
        pltpu.sync_copy(d2b, d2o.at[pl.ds(off, _GCH)])

        for h in hs:
            h.wait()

        @plsc.parallel_loop(0, _GCH, unroll=4)
        def _(r):
            for c in range(D // _L):
                sl = pl.ds(c * _L, _L)
                bufa[r, sl] = bufa[r, sl] + bufb[r, sl]

        pltpu.sync_copy(bufa, gs.at[pl.ds(off, _GCH)])
        return carry

    lax.fori_loop(0, _GNCH, chunk, 0)
    for blk in range(N // BN):
        pltpu.sync_copy(cntb.at[pl.ds(blk * BN, BN)],
                        cnto.at[pl.ds((blk * _NW + wid) * BN, BN)])


def _sc_gather(ta, tb, src, dst, px, py, pz):
    f = pl.kernel(
        _sc_gather_body,
        out_type=[
            jax.ShapeDtypeStruct((_EH, D), jnp.float32),
            jax.ShapeDtypeStruct((_EH,), jnp.float32),
            jax.ShapeDtypeStruct((N * _NW,), jnp.float32),
        ],
        mesh=_sc_mesh(),
        scratch_types=[
            pltpu.VMEM((_GCH,), jnp.int32),
            pltpu.VMEM((_GCH,), jnp.int32),
            pltpu.VMEM((_GCH, D), jnp.float32),
            pltpu.VMEM((_GCH, D), jnp.float32),
            pltpu.VMEM((_GCH,), jnp.float32),
            pltpu.VMEM((N,), jnp.float32),
            pltpu.VMEM((N,), jnp.float32),
            pltpu.VMEM((N,), jnp.float32),
            pltpu.VMEM((N,), jnp.float32),
            pltpu.SemaphoreType.DMA,
        ],
        compiler_params=pltpu.CompilerParams(needs_layout_passes=False),
    )
    return f(ta, tb, src, dst, px, py, pz)


def _sc_scatter_body(msgp, dstr, zer, out, shared, msgbuf, idxv0, idxv1, sem):
    cid = lax.axis_index("c")
    sid = lax.axis_index("s")
    wid = sid * _NC + cid
    rows = pl.ds(sid * _RPT, _RPT)
    tail = pl.ds(_NS * _RPT, _RTAIL)
    pltpu.sync_copy(zer.at[cid, rows], shared.at[rows])

    @pl.when(sid == 0)
    def _():
        pltpu.sync_copy(zer.at[cid, tail], shared.at[tail])

    plsc.subcore_barrier()
    base = wid * _EPW

    def chunk(i, carry):
        off = base + i * _SCH
        pltpu.sync_copy(msgp.at[pl.ds(off, _SCH)], msgbuf)
        (o0, z0), (o1, z1) = _SSUBS
        pltpu.sync_copy(dstr.at[pl.ds(off + o0, z0)], idxv0)
        pltpu.sync_copy(dstr.at[pl.ds(off + o1, z1)], idxv1)
        h0 = pltpu.async_copy(msgbuf.at[pl.ds(o0, z0)],
                              shared.at[idxv0], sem, add=True)
        h1 = pltpu.async_copy(msgbuf.at[pl.ds(o1, z1)],
                              shared.at[idxv1], sem, add=True)
        h0.wait()
        h1.wait()
        return carry

    lax.fori_loop(0, _SNCH, chunk, 0)
    plsc.subcore_barrier()
    pltpu.sync_copy(shared.at[rows], out.at[cid, rows])

    @pl.when(sid == 0)
    def _():
        pltpu.sync_copy(shared.at[tail], out.at[cid, tail])


def _sc_scatter(msgp, dstr, init):
    f = pl.kernel(
        _sc_scatter_body,
        out_type=jax.ShapeDtypeStruct((_NC, N, D), jnp.float32),
        mesh=_sc_mesh(),
        scratch_types=[
            pltpu.MemorySpace.VMEM_SHARED((N, D), jnp.float32),
            pltpu.VMEM((_SCH, D), jnp.float32),
            pltpu.VMEM((_SSUBS[0][1],), jnp.int32),
            pltpu.VMEM((_SSUBS[1][1],), jnp.int32),
            pltpu.SemaphoreType.DMA,
        ],
        compiler_params=pltpu.CompilerParams(needs_layout_passes=False),
    )
    return f(msgp, dstr, init)


def _edge_body(gs_ref, d2_ref, et_ref, emb_ref, w1ct_ref, w1rt_ref,
               w1d_ref, b1_ref, w2t_ref, b2_ref, out_ref):
    gs = gs_ref[...]
    dist = jnp.sqrt(d2_ref[...])
    centers = STEP * lax.broadcasted_iota(jnp.int32, (1, NUM_RBF), 1).astype(jnp.float32)
    diff = dist - centers
    radial = jnp.exp(-GAMMA * diff * diff)
    t = jnp.dot(emb_ref[...], w1ct_ref[...], preferred_element_type=jnp.float32) \
        + b1_ref[...]
    et = et_ref[...]
    tsel = t[0:1, :] * (1.0 - et) + t[1:2, :] * et
    pre1 = (gs + tsel
            + jnp.dot(radial, w1rt_ref[...], preferred_element_type=jnp.float32)
            + dist * w1d_ref[...])
    x = _silu(pre1)
    out_ref[...] = _silu(
        jnp.dot(x, w2t_ref[...], preferred_element_type=jnp.float32)
        + b2_ref[...])


def _edge_mlp(gs, d2, etf, emb, w1ct, w1rt, w1d, b1, w2t, b2):
    grid = _EH // BE
    full = lambda i: (0, 0)
    return pl.pallas_call(
        _edge_body,
        grid=(grid,),
        in_specs=[
            pl.BlockSpec((BE, D), lambda i: (i, 0)),
            pl.BlockSpec((BE, 1), lambda i: (i, 0)),
            pl.BlockSpec((BE, 1), lambda i: (i, 0)),
            pl.BlockSpec((2, D), full),
            pl.BlockSpec((D, D), full),
            pl.BlockSpec((NUM_RBF, D), full),
            pl.BlockSpec((1, D), full),
            pl.BlockSpec((1, D), full),
            pl.BlockSpec((D, D), full),
            pl.BlockSpec((1, D), full),
        ],
        out_specs=pl.BlockSpec((BE, D), lambda i: (i, 0)),
        out_shape=jax.ShapeDtypeStruct((_EH, D), jnp.float32),
    )(gs, d2, etf, emb, w1ct, w1rt, w1d, b1, w2t, b2)


def _node_body(h_ref, s0_ref, s1_ref, cnt_ref, nt_ref, u1at_ref,
               u1bt_ref, c1_ref, u2t_ref, c2_ref, g_ref, bta_ref, out_ref):
    h = h_ref[...]
    s = s0_ref[...] + s1_ref[...]
    cnt = jnp.sum(cnt_ref[0], axis=0, keepdims=True)
    recip = 1.0 / jnp.maximum(cnt, 1.0)
    ii = lax.broadcasted_iota(jnp.int32, (BN, BN), 0)
    jj = lax.broadcasted_iota(jnp.int32, (BN, BN), 1)
    dg = jnp.where(ii == jj, recip, 0.0)
    agg = jnp.dot(dg, s, preferred_element_type=jnp.float32)
    u = _silu(jnp.dot(h, u1at_ref[...], preferred_element_type=jnp.float32)
              + jnp.dot(agg, u1bt_ref[...], preferred_element_type=jnp.float32)
              + c1_ref[...])
    upd = jnp.dot(u, u2t_ref[...], preferred_element_type=jnp.float32) + c2_ref[...]
    pre = h + upd
    mu = jnp.mean(pre, axis=1, keepdims=True)
    cent = pre - mu
    var = jnp.mean(cent * cent, axis=1, keepdims=True)
    ln = cent * lax.rsqrt(var + 1e-05) * g_ref[...] + bta_ref[...]
    out_ref[...] = jnp.where(nt_ref[...] == 1.0, ln, h)


def _node_update(h, s0, s1, cnt, ntf, u1at, u1bt, c1, u2t, c2, g, b):
    grid = N // BN
    full = lambda i: (0, 0)
    return pl.pallas_call(
        _node_body,
        grid=(grid,),
        in_specs=[
            pl.BlockSpec((BN, D), lambda i: (i, 0)),
            pl.BlockSpec((BN, D), lambda i: (i, 0)),
            pl.BlockSpec((BN, D), lambda i: (i, 0)),
            pl.BlockSpec((1, _NW, BN), lambda i: (i, 0, 0)),
            pl.BlockSpec((BN, 1), lambda i: (i, 0)),
            pl.BlockSpec((D, D), full),
            pl.BlockSpec((D, D), full),
            pl.BlockSpec((1, D), full),
            pl.BlockSpec((D, D), full),
            pl.BlockSpec((1, D), full),
            pl.BlockSpec((1, D), full),
            pl.BlockSpec((1, D), full),
        ],
        out_specs=pl.BlockSpec((BN, D), lambda i: (i, 0)),
        out_shape=jax.ShapeDtypeStruct((N, D), jnp.float32),
    )(h, s0, s1, cnt, ntf, u1at, u1bt, c1, u2t, c2, g, b)


def kernel(h, pos, edge_index, edge_type, node_type, emb, W1, b1, W2, b2,
           U1, c1, U2, c2, gamma_ln, beta_ln):
    src32 = edge_index[0].astype(jnp.int32)
    dst32 = edge_index[1].astype(jnp.int32)
    w1at = W1[:, :D].T
    w1bt = W1[:, D:2 * D].T
    w1ct = W1[:, 2 * D:3 * D].T
    w1rt = W1[:, 3 * D:3 * D + NUM_RBF].T
    w1d = W1[:, 3 * D + NUM_RBF][None, :]
    b1r = b1[None, :]
    w2t = W2.T
    b2r = b2[None, :]
    u1at = U1[:, :D].T
    u1bt = U1[:, D:].T
    c1r = c1[None, :]
    u2t = U2.T
    c2r = c2[None, :]
    gr = gamma_ln[None, :]
    br = beta_ln[None, :]
    px = pos[:, 0]
    py = pos[:, 1]
    pz = pos[:, 2]

    ta, tb = _build_tables(h, w1at, w1bt)

    etf = edge_type.astype(jnp.float32)[:, None]

    gs, d2, cntf = _sc_gather(ta, tb, src32, dst32, px, py, pz)
    cnt = cntf.reshape(N // BN, _NW, BN)
    msgp = _edge_mlp(gs, d2[:, None], etf, emb, w1ct, w1rt, w1d, b1r,
                     w2t, b2r)
    zer = jnp.zeros((_NC, N, D), jnp.float32)
    parts = _sc_scatter(msgp, dst32, zer)

    ntf = node_type.astype(jnp.float32)[:, None]
    return _node_update(h, parts[0], parts[1], cnt, ntf, u1at,
                        u1bt, c1r, u2t, c2r, gr, br)


# speedup vs baseline: 1.0472x; 1.0170x over previous
import functools

import jax
import jax.numpy as jnp
from jax import lax
from jax.experimental import pallas as pl
from jax.experimental.pallas import tpu as pltpu
from jax.experimental.pallas import tpu_sc as plsc

N = 10000
E = 320000
D = 128
NUM_RBF = 32
CUTOFF = 6.0
STEP = CUTOFF / (NUM_RBF - 1)
GAMMA = 1.0 / max(STEP * STEP, 1e-06)

BN = 1000
BE = 2000

_NC = 2
_NS = 16
_NW = _NC * _NS
_L = 16

_NH = 1
_EH = E // _NH
_EPW = _EH // _NW

_GCH = 400
_GSUBS = tuple((o, 80) for o in range(0, 400, 80))
_GNCH = _EPW // _GCH

_SCH = 200
_SSUBS = ((0, 104), (104, 96))
_SNCH = _EPW // _SCH
_RPT = 624
_RTAIL = N - _NS * _RPT


def _silu(x):
    return x * (1.0 / (1.0 + jnp.exp(-x)))


def _table_body(h_ref, w1at_ref, w1bt_ref, a_ref, b_ref):
    h = h_ref[...]
    a_ref[...] = jnp.dot(h, w1at_ref[...], preferred_element_type=jnp.float32)
    b_ref[...] = jnp.dot(h, w1bt_ref[...], preferred_element_type=jnp.float32)


def _build_tables(h, w1at, w1bt):
    grid = N // BN
    return pl.pallas_call(
        _table_body,
        grid=(grid,),
        in_specs=[
            pl.BlockSpec((BN, D), lambda i: (i, 0)),
            pl.BlockSpec((D, D), lambda i: (0, 0)),
            pl.BlockSpec((D, D), lambda i: (0, 0)),
        ],
        out_specs=[
            pl.BlockSpec((BN, D), lambda i: (i, 0)),
            pl.BlockSpec((BN, D), lambda i: (i, 0)),
        ],
        out_shape=[
            jax.ShapeDtypeStruct((N, D), jnp.float32),
            jax.ShapeDtypeStruct((N, D), jnp.float32),
        ],
    )(h, w1at, w1bt)


def _sc_mesh():
    return plsc.VectorSubcoreMesh(core_axis_name="c", subcore_axis_name="s")


def _sc_gather_body(ta, tb, srcr, dstr, pxr, pyr, pzr,
                    ga, gb, d2o, cnto,
                    idxs, idxd, buf, d2b, cntb, px, py, pz, sem):
    cid = lax.axis_index("c")
    sid = lax.axis_index("s")
    wid = sid * _NC + cid
    base = wid * _EPW

    pltpu.sync_copy(pxr, px)
    pltpu.sync_copy(pyr, py)
    pltpu.sync_copy(pzr, pz)
    pltpu.sync_copy(srcr.at[pl.ds(base, _EPW)], idxs)
    pltpu.sync_copy(dstr.at[pl.ds(base, _EPW)], idxd)

    zero16 = jnp.zeros((_L,), jnp.float32)

    def zinit(r, carry):
        cntb[pl.ds(r * _L, _L)] = zero16
        return carry

    lax.fori_loop(0, N // _L, zinit, 0)

    one16 = jnp.ones((_L,), jnp.float32)

    def chunk(i, carry):
        off = base + i * _GCH
        coff = i * _GCH
        ha = []
        for (o, sz) in _GSUBS:
            ha.append(pltpu.async_copy(ta.at[idxs.at[pl.ds(coff + o, sz)]],
                                       buf.at[pl.ds(o, sz)], sem))

        def dcomp(k, c2):
            s16 = pl.ds(coff + k * _L, _L)
            iv_s = idxs[s16]
            iv_d = idxd[s16]
            dx = plsc.load_gather(px, [iv_s]) - plsc.load_gather(px, [iv_d])
            dy = plsc.load_gather(py, [iv_s]) - plsc.load_gather(py, [iv_d])
            dz = plsc.load_gather(pz, [iv_s]) - plsc.load_gather(pz, [iv_d])
            d2b[pl.ds(k * _L, _L)] = dx * dx + dy * dy + dz * dz
            plsc.addupdate_scatter(cntb, [iv_d], one16)
            return c2

        lax.fori_loop(0, _GCH // _L, dcomp, 0)
        pltpu.sync_copy(d2b, d2o.at[pl.ds(off, _GCH)])

        for h in ha:
            h.wait()
        pltpu.sync_copy(buf, ga.at[pl.ds(off, _GCH)])
        hb = []
        for (o, sz) in _GSUBS:
            hb.append(pltpu.async_copy(tb.at[idxd.at[pl.ds(coff + o, sz)]],
                                       buf.at[pl.ds(o, sz)], sem))
        for h in hb:
            h.wait()
        pltpu.sync_copy(buf, gb.at[pl.ds(off, _GCH)])
        return carry

    lax.fori_loop(0, _GNCH, chunk, 0)
    for blk in range(N // BN):
        pltpu.sync_copy(cntb.at[pl.ds(blk * BN, BN)],
                        cnto.at[pl.ds((blk * _NW + wid) * BN, BN)])


def _sc_gather(ta, tb, src, dst, px, py, pz):
    f = pl.kernel(
        _sc_gather_body,
        out_type=[
            jax.ShapeDtypeStruct((_EH, D), jnp.float32),
            jax.ShapeDtypeStruct((_EH, D), jnp.float32),
            jax.ShapeDtypeStruct((_EH,), jnp.float32),
            jax.ShapeDtypeStruct((N * _NW,), jnp.float32),
        ],
        mesh=_sc_mesh(),
        scratch_types=[
            pltpu.VMEM((_EPW,), jnp.int32),
            pltpu.VMEM((_EPW,), jnp.int32),
            pltpu.VMEM((_GCH, D), jnp.float32),
            pltpu.VMEM((_GCH,), jnp.float32),
            pltpu.VMEM((N,), jnp.float32),
            pltpu.VMEM((N,), jnp.float32),
            pltpu.VMEM((N,), jnp.float32),
            pltpu.VMEM((N,), jnp.float32),
            pltpu.SemaphoreType.DMA,
        ],
        compiler_params=pltpu.CompilerParams(needs_layout_passes=False),
    )
    return f(ta, tb, src, dst, px, py, pz)


def _sc_scatter_body(msgp, dstr, zer, out, shared, msgbuf, idxv0, idxv1, sem):
    cid = lax.axis_index("c")
    sid = lax.axis_index("s")
    wid = sid * _NC + cid
    rows = pl.ds(sid * _RPT, _RPT)
    tail = pl.ds(_NS * _RPT, _RTAIL)
    pltpu.sync_copy(zer.at[cid, rows], shared.at[rows])

    @pl.when(sid == 0)
    def _():
        pltpu.sync_copy(zer.at[cid, tail], shared.at[tail])

    plsc.subcore_barrier()
    base = wid * _EPW

    def chunk(i, carry):
        off = base + i * _SCH
        pltpu.sync_copy(msgp.at[pl.ds(off, _SCH)], msgbuf)
        (o0, z0), (o1, z1) = _SSUBS
        pltpu.sync_copy(dstr.at[pl.ds(off + o0, z0)], idxv0)
        pltpu.sync_copy(dstr.at[pl.ds(off + o1, z1)], idxv1)
        h0 = pltpu.async_copy(msgbuf.at[pl.ds(o0, z0)],
                              shared.at[idxv0], sem, add=True)
        h1 = pltpu.async_copy(msgbuf.at[pl.ds(o1, z1)],
                              shared.at[idxv1], sem, add=True)
        h0.wait()
        h1.wait()
        return carry

    lax.fori_loop(0, _SNCH, chunk, 0)
    plsc.subcore_barrier()
    pltpu.sync_copy(shared.at[rows], out.at[cid, rows])

    @pl.when(sid == 0)
    def _():
        pltpu.sync_copy(shared.at[tail], out.at[cid, tail])


def _sc_scatter(msgp, dstr, init):
    f = pl.kernel(
        _sc_scatter_body,
        out_type=jax.ShapeDtypeStruct((_NC, N, D), jnp.float32),
        mesh=_sc_mesh(),
        scratch_types=[
            pltpu.MemorySpace.VMEM_SHARED((N, D), jnp.float32),
            pltpu.VMEM((_SCH, D), jnp.float32),
            pltpu.VMEM((_SSUBS[0][1],), jnp.int32),
            pltpu.VMEM((_SSUBS[1][1],), jnp.int32),
            pltpu.SemaphoreType.DMA,
        ],
        compiler_params=pltpu.CompilerParams(needs_layout_passes=False),
    )
    return f(msgp, dstr, init)


def _edge_body(ga_ref, gb_ref, d2_ref, et_ref, emb_ref, w1ct_ref, w1rt_ref,
               w1d_ref, b1_ref, w2t_ref, b2_ref, out_ref):
    gs = ga_ref[...] + gb_ref[...]
    dist = jnp.sqrt(d2_ref[...])
    centers = STEP * lax.broadcasted_iota(jnp.int32, (1, NUM_RBF), 1).astype(jnp.float32)
    diff = dist - centers
    radial = jnp.exp(-GAMMA * diff * diff)
    t = jnp.dot(emb_ref[...], w1ct_ref[...], preferred_element_type=jnp.float32) \
        + b1_ref[...]
    et = et_ref[...]
    tsel = t[0:1, :] * (1.0 - et) + t[1:2, :] * et
    pre1 = (gs + tsel
            + jnp.dot(radial, w1rt_ref[...], preferred_element_type=jnp.float32)
            + dist * w1d_ref[...])
    x = _silu(pre1)
    out_ref[...] = _silu(
        jnp.dot(x, w2t_ref[...], preferred_element_type=jnp.float32)
        + b2_ref[...])


def _edge_mlp(ga, gb, d2, etf, emb, w1ct, w1rt, w1d, b1, w2t, b2):
    grid = _EH // BE
    full = lambda i: (0, 0)
    return pl.pallas_call(
        _edge_body,
        grid=(grid,),
        in_specs=[
            pl.BlockSpec((BE, D), lambda i: (i, 0)),
            pl.BlockSpec((BE, D), lambda i: (i, 0)),
            pl.BlockSpec((BE, 1), lambda i: (i, 0)),
            pl.BlockSpec((BE, 1), lambda i: (i, 0)),
            pl.BlockSpec((2, D), full),
            pl.BlockSpec((D, D), full),
            pl.BlockSpec((NUM_RBF, D), full),
            pl.BlockSpec((1, D), full),
            pl.BlockSpec((1, D), full),
            pl.BlockSpec((D, D), full),
            pl.BlockSpec((1, D), full),
        ],
        out_specs=pl.BlockSpec((BE, D), lambda i: (i, 0)),
        out_shape=jax.ShapeDtypeStruct((_EH, D), jnp.float32),
    )(ga, gb, d2, etf, emb, w1ct, w1rt, w1d, b1, w2t, b2)


def _node_body(h_ref, s0_ref, s1_ref, cnt_ref, nt_ref, u1at_ref,
               u1bt_ref, c1_ref, u2t_ref, c2_ref, g_ref, bta_ref, out_ref):
    h = h_ref[...]
    s = s0_ref[...] + s1_ref[...]
    cnt = jnp.sum(cnt_ref[0], axis=0, keepdims=True)
    recip = 1.0 / jnp.maximum(cnt, 1.0)
    ii = lax.broadcasted_iota(jnp.int32, (BN, BN), 0)
    jj = lax.broadcasted_iota(jnp.int32, (BN, BN), 1)
    dg = jnp.where(ii == jj, recip, 0.0)
    agg = jnp.dot(dg, s, preferred_element_type=jnp.float32)
    u = _silu(jnp.dot(h, u1at_ref[...], preferred_element_type=jnp.float32)
              + jnp.dot(agg, u1bt_ref[...], preferred_element_type=jnp.float32)
              + c1_ref[...])
    upd = jnp.dot(u, u2t_ref[...], preferred_element_type=jnp.float32) + c2_ref[...]
    pre = h + upd
    mu = jnp.mean(pre, axis=1, keepdims=True)
    cent = pre - mu
    var = jnp.mean(cent * cent, axis=1, keepdims=True)
    ln = cent * lax.rsqrt(var + 1e-05) * g_ref[...] + bta_ref[...]
    out_ref[...] = jnp.where(nt_ref[...] == 1.0, ln, h)


def _node_update(h, s0, s1, cnt, ntf, u1at, u1bt, c1, u2t, c2, g, b):
    grid = N // BN
    full = lambda i: (0, 0)
    return pl.pallas_call(
        _node_body,
        grid=(grid,),
        in_specs=[
            pl.BlockSpec((BN, D), lambda i: (i, 0)),
            pl.BlockSpec((BN, D), lambda i: (i, 0)),
            pl.BlockSpec((BN, D), lambda i: (i, 0)),
            pl.BlockSpec((1, _NW, BN), lambda i: (i, 0, 0)),
            pl.BlockSpec((BN, 1), lambda i: (i, 0)),
            pl.BlockSpec((D, D), full),
            pl.BlockSpec((D, D), full),
            pl.BlockSpec((1, D), full),
            pl.BlockSpec((D, D), full),
            pl.BlockSpec((1, D), full),
            pl.BlockSpec((1, D), full),
            pl.BlockSpec((1, D), full),
        ],
        out_specs=pl.BlockSpec((BN, D), lambda i: (i, 0)),
        out_shape=jax.ShapeDtypeStruct((N, D), jnp.float32),
    )(h, s0, s1, cnt, ntf, u1at, u1bt, c1, u2t, c2, g, b)


def kernel(h, pos, edge_index, edge_type, node_type, emb, W1, b1, W2, b2,
           U1, c1, U2, c2, gamma_ln, beta_ln):
    src32 = edge_index[0].astype(jnp.int32)
    dst32 = edge_index[1].astype(jnp.int32)
    w1at = W1[:, :D].T
    w1bt = W1[:, D:2 * D].T
    w1ct = W1[:, 2 * D:3 * D].T
    w1rt = W1[:, 3 * D:3 * D + NUM_RBF].T
    w1d = W1[:, 3 * D + NUM_RBF][None, :]
    b1r = b1[None, :]
    w2t = W2.T
    b2r = b2[None, :]
    u1at = U1[:, :D].T
    u1bt = U1[:, D:].T
    c1r = c1[None, :]
    u2t = U2.T
    c2r = c2[None, :]
    gr = gamma_ln[None, :]
    br = beta_ln[None, :]
    px = pos[:, 0]
    py = pos[:, 1]
    pz = pos[:, 2]

    ta, tb = _build_tables(h, w1at, w1bt)

    etf = edge_type.astype(jnp.float32)[:, None]

    ga, gb, d2, cntf = _sc_gather(ta, tb, src32, dst32, px, py, pz)
    cnt = cntf.reshape(N // BN, _NW, BN)
    msgp = _edge_mlp(ga, gb, d2[:, None], etf, emb, w1ct, w1rt, w1d, b1r,
                     w2t, b2r)
    zer = jnp.zeros((_NC, N, D), jnp.float32)
    parts = _sc_scatter(msgp, dst32, zer)

    ntf = node_type.astype(jnp.float32)[:, None]
    return _node_update(h, parts[0], parts[1], cnt, ntf, u1at,
                        u1bt, c1r, u2t, c2r, gr, br)


# speedup vs baseline: 1.0657x; 1.0177x over previous
import functools

import jax
import jax.numpy as jnp
from jax import lax
from jax.experimental import pallas as pl
from jax.experimental.pallas import tpu as pltpu
from jax.experimental.pallas import tpu_sc as plsc

N = 10000
E = 320000
D = 128
NUM_RBF = 32
CUTOFF = 6.0
STEP = CUTOFF / (NUM_RBF - 1)
GAMMA = 1.0 / max(STEP * STEP, 1e-06)

BN = 1000
BE = 2000

_NC = 2
_NS = 16
_NW = _NC * _NS
_L = 16

_NH = 1
_EH = E // _NH
_EPW = _EH // _NW

_GCH = 400
_GSUBS = tuple((o, 80) for o in range(0, 400, 80))
_GNCH = _EPW // _GCH

_SCH = 200
_SSUB = 100
_SNSUB = _SCH // _SSUB
_SNCH = _EPW // _SCH
_RPT = 624
_RTAIL = N - _NS * _RPT


def _silu(x):
    return x * (1.0 / (1.0 + jnp.exp(-x)))


def _table_body(h_ref, w1at_ref, w1bt_ref, a_ref, b_ref):
    h = h_ref[...]
    a_ref[...] = jnp.dot(h, w1at_ref[...], preferred_element_type=jnp.float32)
    b_ref[...] = jnp.dot(h, w1bt_ref[...], preferred_element_type=jnp.float32)


def _build_tables(h, w1at, w1bt):
    grid = N // BN
    return pl.pallas_call(
        _table_body,
        grid=(grid,),
        in_specs=[
            pl.BlockSpec((BN, D), lambda i: (i, 0)),
            pl.BlockSpec((D, D), lambda i: (0, 0)),
            pl.BlockSpec((D, D), lambda i: (0, 0)),
        ],
        out_specs=[
            pl.BlockSpec((BN, D), lambda i: (i, 0)),
            pl.BlockSpec((BN, D), lambda i: (i, 0)),
        ],
        out_shape=[
            jax.ShapeDtypeStruct((N, D), jnp.float32),
            jax.ShapeDtypeStruct((N, D), jnp.float32),
        ],
    )(h, w1at, w1bt)


def _sc_mesh():
    return plsc.VectorSubcoreMesh(core_axis_name="c", subcore_axis_name="s")


def _sc_gather_body(ta, tb, srcr, dstr, pxr, pyr, pzr,
                    ga, gb, d2o, cnto,
                    idxs, idxd, buf, d2b, cntb, px, py, pz, sem):
    cid = lax.axis_index("c")
    sid = lax.axis_index("s")
    wid = sid * _NC + cid
    base = wid * _EPW

    pltpu.sync_copy(pxr, px)
    pltpu.sync_copy(pyr, py)
    pltpu.sync_copy(pzr, pz)
    pltpu.sync_copy(srcr.at[pl.ds(base, _EPW)], idxs)
    pltpu.sync_copy(dstr.at[pl.ds(base, _EPW)], idxd)

    zero16 = jnp.zeros((_L,), jnp.float32)

    def zinit(r, carry):
        cntb[pl.ds(r * _L, _L)] = zero16
        return carry

    lax.fori_loop(0, N // _L, zinit, 0)

    one16 = jnp.ones((_L,), jnp.float32)

    def chunk(i, carry):
        off = base + i * _GCH
        coff = i * _GCH
        ha = []
        for (o, sz) in _GSUBS:
            ha.append(pltpu.async_copy(ta.at[idxs.at[pl.ds(coff + o, sz)]],
                                       buf.at[pl.ds(o, sz)], sem))

        def dcomp(k, c2):
            s16 = pl.ds(coff + k * _L, _L)
            iv_s = idxs[s16]
            iv_d = idxd[s16]
            dx = plsc.load_gather(px, [iv_s]) - plsc.load_gather(px, [iv_d])
            dy = plsc.load_gather(py, [iv_s]) - plsc.load_gather(py, [iv_d])
            dz = plsc.load_gather(pz, [iv_s]) - plsc.load_gather(pz, [iv_d])
            d2b[pl.ds(k * _L, _L)] = dx * dx + dy * dy + dz * dz
            plsc.addupdate_scatter(cntb, [iv_d], one16)
            return c2

        lax.fori_loop(0, _GCH // _L, dcomp, 0)
        pltpu.sync_copy(d2b, d2o.at[pl.ds(off, _GCH)])

        for h in ha:
            h.wait()
        pltpu.sync_copy(buf, ga.at[pl.ds(off, _GCH)])
        hb = []
        for (o, sz) in _GSUBS:
            hb.append(pltpu.async_copy(tb.at[idxd.at[pl.ds(coff + o, sz)]],
                                       buf.at[pl.ds(o, sz)], sem))
        for h in hb:
            h.wait()
        pltpu.sync_copy(buf, gb.at[pl.ds(off, _GCH)])
        return carry

    lax.fori_loop(0, _GNCH, chunk, 0)
    for blk in range(N // BN):
        pltpu.sync_copy(cntb.at[pl.ds(blk * BN, BN)],
                        cnto.at[pl.ds((blk * _NW + wid) * BN, BN)])


def _sc_gather(ta, tb, src, dst, px, py, pz):
    f = pl.kernel(
        _sc_gather_body,
        out_type=[
            jax.ShapeDtypeStruct((_EH, D), jnp.float32),
            jax.ShapeDtypeStruct((_EH, D), jnp.float32),
            jax.ShapeDtypeStruct((_EH,), jnp.float32),
            jax.ShapeDtypeStruct((N * _NW,), jnp.float32),
        ],
        mesh=_sc_mesh(),
        scratch_types=[
            pltpu.VMEM((_EPW,), jnp.int32),
            pltpu.VMEM((_EPW,), jnp.int32),
            pltpu.VMEM((_GCH, D), jnp.float32),
            pltpu.VMEM((_GCH,), jnp.float32),
            pltpu.VMEM((N,), jnp.float32),
            pltpu.VMEM((N,), jnp.float32),
            pltpu.VMEM((N,), jnp.float32),
            pltpu.VMEM((N,), jnp.float32),
            pltpu.SemaphoreType.DMA,
        ],
        compiler_params=pltpu.CompilerParams(needs_layout_passes=False),
    )
    return f(ta, tb, src, dst, px, py, pz)


def _sc_scatter_body(msgp, dst2, zer, out, shared, msgbuf, idxv, sem):
    cid = lax.axis_index("c")
    sid = lax.axis_index("s")
    wid = sid * _NC + cid
    rows = pl.ds(sid * _RPT, _RPT)
    tail = pl.ds(_NS * _RPT, _RTAIL)
    pltpu.sync_copy(zer.at[cid, rows], shared.at[rows])

    @pl.when(sid == 0)
    def _():
        pltpu.sync_copy(zer.at[cid, tail], shared.at[tail])

    plsc.subcore_barrier()
    base = wid * _EPW

    def chunk(i, carry):
        off = base + i * _SCH
        pltpu.sync_copy(msgp.at[pl.ds(off, _SCH)], msgbuf)
        row0 = wid * (_EPW // _SSUB) + i * _SNSUB
        pltpu.sync_copy(dst2.at[pl.ds(row0, _SNSUB)], idxv)
        hs = []
        for j in range(_SNSUB):
            hs.append(pltpu.async_copy(
                msgbuf.at[pl.ds(j * _SSUB, _SSUB)],
                shared.at[idxv.at[j]], sem, add=True))
        for h in hs:
            h.wait()
        return carry

    lax.fori_loop(0, _SNCH, chunk, 0)
    plsc.subcore_barrier()
    pltpu.sync_copy(shared.at[rows], out.at[cid, rows])

    @pl.when(sid == 0)
    def _():
        pltpu.sync_copy(shared.at[tail], out.at[cid, tail])


def _sc_scatter(msgp, dst2, init):
    f = pl.kernel(
        _sc_scatter_body,
        out_type=jax.ShapeDtypeStruct((_NC, N, D), jnp.float32),
        mesh=_sc_mesh(),
        scratch_types=[
            pltpu.MemorySpace.VMEM_SHARED((N, D), jnp.float32),
            pltpu.VMEM((_SCH, D), jnp.float32),
            pltpu.VMEM((_SNSUB, _SSUB), jnp.int32),
            pltpu.SemaphoreType.DMA,
        ],
        compiler_params=pltpu.CompilerParams(needs_layout_passes=False),
    )
    return f(msgp, dst2, init)


def _edge_body(ga_ref, gb_ref, d2_ref, et_ref, emb_ref, w1ct_ref, w1rt_ref,
               w1d_ref, b1_ref, w2t_ref, b2_ref, out_ref):
    gs = ga_ref[...] + gb_ref[...]
    dist = jnp.sqrt(d2_ref[...])
    centers = STEP * lax.broadcasted_iota(jnp.int32, (1, NUM_RBF), 1).astype(jnp.float32)
    diff = dist - centers
    radial = jnp.exp(-GAMMA * diff * diff)
    t = jnp.dot(emb_ref[...], w1ct_ref[...], preferred_element_type=jnp.float32) \
        + b1_ref[...]
    et = et_ref[...]
    tsel = t[0:1, :] * (1.0 - et) + t[1:2, :] * et
    pre1 = (gs + tsel
            + jnp.dot(radial, w1rt_ref[...], preferred_element_type=jnp.float32)
            + dist * w1d_ref[...])
    x = _silu(pre1)
    out_ref[...] = _silu(
        jnp.dot(x, w2t_ref[...], preferred_element_type=jnp.float32)
        + b2_ref[...])


def _edge_mlp(ga, gb, d2, etf, emb, w1ct, w1rt, w1d, b1, w2t, b2):
    grid = _EH // BE
    full = lambda i: (0, 0)
    return pl.pallas_call(
        _edge_body,
        grid=(grid,),
        in_specs=[
            pl.BlockSpec((BE, D), lambda i: (i, 0)),
            pl.BlockSpec((BE, D), lambda i: (i, 0)),
            pl.BlockSpec((BE, 1), lambda i: (i, 0)),
            pl.BlockSpec((BE, 1), lambda i: (i, 0)),
            pl.BlockSpec((2, D), full),
            pl.BlockSpec((D, D), full),
            pl.BlockSpec((NUM_RBF, D), full),
            pl.BlockSpec((1, D), full),
            pl.BlockSpec((1, D), full),
            pl.BlockSpec((D, D), full),
            pl.BlockSpec((1, D), full),
        ],
        out_specs=pl.BlockSpec((BE, D), lambda i: (i, 0)),
        out_shape=jax.ShapeDtypeStruct((_EH, D), jnp.float32),
    )(ga, gb, d2, etf, emb, w1ct, w1rt, w1d, b1, w2t, b2)


def _node_body(h_ref, s0_ref, s1_ref, cnt_ref, nt_ref, u1at_ref,
               u1bt_ref, c1_ref, u2t_ref, c2_ref, g_ref, bta_ref, out_ref):
    h = h_ref[...]
    s = s0_ref[...] + s1_ref[...]
    cnt = jnp.sum(cnt_ref[0], axis=0, keepdims=True)
    recip = 1.0 / jnp.maximum(cnt, 1.0)
    ii = lax.broadcasted_iota(jnp.int32, (BN, BN), 0)
    jj = lax.broadcasted_iota(jnp.int32, (BN, BN), 1)
    dg = jnp.where(ii == jj, recip, 0.0)
    agg = jnp.dot(dg, s, preferred_element_type=jnp.float32)
    u = _silu(jnp.dot(h, u1at_ref[...], preferred_element_type=jnp.float32)
              + jnp.dot(agg, u1bt_ref[...], preferred_element_type=jnp.float32)
              + c1_ref[...])
    upd = jnp.dot(u, u2t_ref[...], preferred_element_type=jnp.float32) + c2_ref[...]
    pre = h + upd
    mu = jnp.mean(pre, axis=1, keepdims=True)
    cent = pre - mu
    var = jnp.mean(cent * cent, axis=1, keepdims=True)
    ln = cent * lax.rsqrt(var + 1e-05) * g_ref[...] + bta_ref[...]
    out_ref[...] = jnp.where(nt_ref[...] == 1.0, ln, h)


def _node_update(h, s0, s1, cnt, ntf, u1at, u1bt, c1, u2t, c2, g, b):
    grid = N // BN
    full = lambda i: (0, 0)
    return pl.pallas_call(
        _node_body,
        grid=(grid,),
        in_specs=[
            pl.BlockSpec((BN, D), lambda i: (i, 0)),
            pl.BlockSpec((BN, D), lambda i: (i, 0)),
            pl.BlockSpec((BN, D), lambda i: (i, 0)),
            pl.BlockSpec((1, _NW, BN), lambda i: (i, 0, 0)),
            pl.BlockSpec((BN, 1), lambda i: (i, 0)),
            pl.BlockSpec((D, D), full),
            pl.BlockSpec((D, D), full),
            pl.BlockSpec((1, D), full),
            pl.BlockSpec((D, D), full),
            pl.BlockSpec((1, D), full),
            pl.BlockSpec((1, D), full),
            pl.BlockSpec((1, D), full),
        ],
        out_specs=pl.BlockSpec((BN, D), lambda i: (i, 0)),
        out_shape=jax.ShapeDtypeStruct((N, D), jnp.float32),
    )(h, s0, s1, cnt, ntf, u1at, u1bt, c1, u2t, c2, g, b)


def kernel(h, pos, edge_index, edge_type, node_type, emb, W1, b1, W2, b2,
           U1, c1, U2, c2, gamma_ln, beta_ln):
    src32 = edge_index[0].astype(jnp.int32)
    dst32 = edge_index[1].astype(jnp.int32)
    w1at = W1[:, :D].T
    w1bt = W1[:, D:2 * D].T
    w1ct = W1[:, 2 * D:3 * D].T
    w1rt = W1[:, 3 * D:3 * D + NUM_RBF].T
    w1d = W1[:, 3 * D + NUM_RBF][None, :]
    b1r = b1[None, :]
    w2t = W2.T
    b2r = b2[None, :]
    u1at = U1[:, :D].T
    u1bt = U1[:, D:].T
    c1r = c1[None, :]
    u2t = U2.T
    c2r = c2[None, :]
    gr = gamma_ln[None, :]
    br = beta_ln[None, :]
    px = pos[:, 0]
    py = pos[:, 1]
    pz = pos[:, 2]

    ta, tb = _build_tables(h, w1at, w1bt)

    etf = edge_type.astype(jnp.float32)[:, None]

    ga, gb, d2, cntf = _sc_gather(ta, tb, src32, dst32, px, py, pz)
    cnt = cntf.reshape(N // BN, _NW, BN)
    msgp = _edge_mlp(ga, gb, d2[:, None], etf, emb, w1ct, w1rt, w1d, b1r,
                     w2t, b2r)
    zer = jnp.zeros((_NC, N, D), jnp.float32)
    dst2 = dst32.reshape(E // _SSUB, _SSUB)
    parts = _sc_scatter(msgp, dst2, zer)

    ntf = node_type.astype(jnp.float32)[:, None]
    return _node_update(h, parts[0], parts[1], cnt, ntf, u1at,
                        u1bt, c1r, u2t, c2r, gr, br)


# speedup vs baseline: 1.0674x; 1.0016x over previous
import functools

import jax
import jax.numpy as jnp
from jax import lax
from jax.experimental import pallas as pl
from jax.experimental.pallas import tpu as pltpu
from jax.experimental.pallas import tpu_sc as plsc

N = 10000
E = 320000
D = 128
NUM_RBF = 32
CUTOFF = 6.0
STEP = CUTOFF / (NUM_RBF - 1)
GAMMA = 1.0 / max(STEP * STEP, 1e-06)

BN = 1000
BE = 2000

_NC = 2
_NS = 16
_NW = _NC * _NS
_L = 16

_NH = 1
_EH = E // _NH
_EPW = _EH // _NW

_GCH = 400
_GSUBS = tuple((o, 80) for o in range(0, 400, 80))
_GNCH = _EPW // _GCH

_SCH = 200
_SSUB = 100
_SNSUB = _SCH // _SSUB
_SNCH = _EPW // _SCH
_RPT = 624
_RTAIL = N - _NS * _RPT


def _silu(x):
    return x * (1.0 / (1.0 + jnp.exp(-x)))


def _table_body(h_ref, w1at_ref, w1bt_ref, a_ref, b_ref):
    h = h_ref[...]
    a_ref[...] = jnp.dot(h, w1at_ref[...], preferred_element_type=jnp.float32)
    b_ref[...] = jnp.dot(h, w1bt_ref[...], preferred_element_type=jnp.float32)


def _build_tables(h, w1at, w1bt):
    grid = N // BN
    return pl.pallas_call(
        _table_body,
        grid=(grid,),
        in_specs=[
            pl.BlockSpec((BN, D), lambda i: (i, 0)),
            pl.BlockSpec((D, D), lambda i: (0, 0)),
            pl.BlockSpec((D, D), lambda i: (0, 0)),
        ],
        out_specs=[
            pl.BlockSpec((BN, D), lambda i: (i, 0)),
            pl.BlockSpec((BN, D), lambda i: (i, 0)),
        ],
        out_shape=[
            jax.ShapeDtypeStruct((N, D), jnp.float32),
            jax.ShapeDtypeStruct((N, D), jnp.float32),
        ],
    )(h, w1at, w1bt)


def _sc_mesh():
    return plsc.VectorSubcoreMesh(core_axis_name="c", subcore_axis_name="s")


def _sc_gather_body(ta, tb, srcr, dstr, pxr, pyr, pzr,
                    ga, gb, d2o, cnto,
                    idxs, idxd, buf, d2b, cntb, px, py, pz, sem):
    cid = lax.axis_index("c")
    sid = lax.axis_index("s")
    wid = sid * _NC + cid
    base = wid * _EPW

    pltpu.sync_copy(pxr, px)
    pltpu.sync_copy(pyr, py)
    pltpu.sync_copy(pzr, pz)
    pltpu.sync_copy(srcr.at[pl.ds(base, _EPW)], idxs)
    pltpu.sync_copy(dstr.at[pl.ds(base, _EPW)], idxd)

    zero16 = jnp.zeros((_L,), jnp.float32)

    def zinit(r, carry):
        cntb[pl.ds(r * _L, _L)] = zero16
        return carry

    lax.fori_loop(0, N // _L, zinit, 0)

    one16 = jnp.ones((_L,), jnp.float32)

    def chunk(i, carry):
        off = base + i * _GCH
        coff = i * _GCH
        ha = []
        for (o, sz) in _GSUBS:
            ha.append(pltpu.async_copy(ta.at[idxs.at[pl.ds(coff + o, sz)]],
                                       buf.at[pl.ds(o, sz)], sem))

        def dcomp(k, c2):
            s16 = pl.ds(coff + k * _L, _L)
            iv_s = idxs[s16]
            iv_d = idxd[s16]
            dx = plsc.load_gather(px, [iv_s]) - plsc.load_gather(px, [iv_d])
            dy = plsc.load_gather(py, [iv_s]) - plsc.load_gather(py, [iv_d])
            dz = plsc.load_gather(pz, [iv_s]) - plsc.load_gather(pz, [iv_d])
            d2b[pl.ds(k * _L, _L)] = dx * dx + dy * dy + dz * dz
            plsc.addupdate_scatter(cntb, [iv_d], one16)
            return c2

        lax.fori_loop(0, _GCH // _L, dcomp, 0)
        pltpu.sync_copy(d2b, d2o.at[pl.ds(off, _GCH)])

        for h in ha:
            h.wait()
        pltpu.sync_copy(buf, ga.at[pl.ds(off, _GCH)])
        hb = []
        for (o, sz) in _GSUBS:
            hb.append(pltpu.async_copy(tb.at[idxd.at[pl.ds(coff + o, sz)]],
                                       buf.at[pl.ds(o, sz)], sem))
        for h in hb:
            h.wait()
        pltpu.sync_copy(buf, gb.at[pl.ds(off, _GCH)])
        return carry

    lax.fori_loop(0, _GNCH, chunk, 0)
    for blk in range(N // BN):
        pltpu.sync_copy(cntb.at[pl.ds(blk * BN, BN)],
                        cnto.at[pl.ds((blk * _NW + wid) * BN, BN)])


def _sc_gather(ta, tb, src, dst, px, py, pz):
    f = pl.kernel(
        _sc_gather_body,
        out_type=[
            jax.ShapeDtypeStruct((_EH, D), jnp.float32),
            jax.ShapeDtypeStruct((_EH, D), jnp.float32),
            jax.ShapeDtypeStruct((_EH,), jnp.float32),
            jax.ShapeDtypeStruct((N * _NW,), jnp.float32),
        ],
        mesh=_sc_mesh(),
        scratch_types=[
            pltpu.VMEM((_EPW,), jnp.int32),
            pltpu.VMEM((_EPW,), jnp.int32),
            pltpu.VMEM((_GCH, D), jnp.float32),
            pltpu.VMEM((_GCH,), jnp.float32),
            pltpu.VMEM((N,), jnp.float32),
            pltpu.VMEM((N,), jnp.float32),
            pltpu.VMEM((N,), jnp.float32),
            pltpu.VMEM((N,), jnp.float32),
            pltpu.SemaphoreType.DMA,
        ],
        compiler_params=pltpu.CompilerParams(needs_layout_passes=False),
    )
    return f(ta, tb, src, dst, px, py, pz)


def _sc_scatter_body(msgp, dst2, zer, out, shared, msgbuf, idxv, sem):
    cid = lax.axis_index("c")
    sid = lax.axis_index("s")
    wid = sid * _NC + cid
    rows = pl.ds(sid * _RPT, _RPT)
    tail = pl.ds(_NS * _RPT, _RTAIL)
    pltpu.sync_copy(zer.at[rows], shared.at[rows])

    @pl.when(sid == 0)
    def _():
        pltpu.sync_copy(zer.at[tail], shared.at[tail])

    plsc.subcore_barrier()
    base = wid * _EPW

    def chunk(i, carry):
        off = base + i * _SCH
        pltpu.sync_copy(msgp.at[pl.ds(off, _SCH)], msgbuf)
        row0 = wid * (_EPW // _SSUB) + i * _SNSUB
        pltpu.sync_copy(dst2.at[pl.ds(row0, _SNSUB)], idxv)
        hs = []
        for j in range(_SNSUB):
            hs.append(pltpu.async_copy(
                msgbuf.at[pl.ds(j * _SSUB, _SSUB)],
                shared.at[idxv.at[j]], sem, add=True))
        for h in hs:
            h.wait()
        return carry

    lax.fori_loop(0, _SNCH, chunk, 0)
    plsc.subcore_barrier()
    pltpu.sync_copy(shared.at[rows], out.at[cid, rows])

    @pl.when(sid == 0)
    def _():
        pltpu.sync_copy(shared.at[tail], out.at[cid, tail])


def _sc_scatter(msgp, dst2, init):
    f = pl.kernel(
        _sc_scatter_body,
        out_type=jax.ShapeDtypeStruct((_NC, N, D), jnp.float32),
        mesh=_sc_mesh(),
        scratch_types=[
            pltpu.MemorySpace.VMEM_SHARED((N, D), jnp.float32),
            pltpu.VMEM((_SCH, D), jnp.float32),
            pltpu.VMEM((_SNSUB, _SSUB), jnp.int32),
            pltpu.SemaphoreType.DMA,
        ],
        compiler_params=pltpu.CompilerParams(needs_layout_passes=False),
    )
    return f(msgp, dst2, init)


def _edge_body(ga_ref, gb_ref, d2_ref, et_ref, emb_ref, w1ct_ref, w1rt_ref,
               w1d_ref, b1_ref, w2t_ref, b2_ref, out_ref):
    gs = ga_ref[...] + gb_ref[...]
    dist = jnp.sqrt(d2_ref[...])
    centers = STEP * lax.broadcasted_iota(jnp.int32, (1, NUM_RBF), 1).astype(jnp.float32)
    diff = dist - centers
    radial = jnp.exp(-GAMMA * diff * diff)
    t = jnp.dot(emb_ref[...], w1ct_ref[...], preferred_element_type=jnp.float32) \
        + b1_ref[...]
    et = et_ref[...]
    tsel = t[0:1, :] * (1.0 - et) + t[1:2, :] * et
    pre1 = (gs + tsel
            + jnp.dot(radial, w1rt_ref[...], preferred_element_type=jnp.float32)
            + dist * w1d_ref[...])
    x = _silu(pre1)
    out_ref[...] = _silu(
        jnp.dot(x, w2t_ref[...], preferred_element_type=jnp.float32)
        + b2_ref[...])


def _edge_mlp(ga, gb, d2, etf, emb, w1ct, w1rt, w1d, b1, w2t, b2):
    grid = _EH // BE
    full = lambda i: (0, 0)
    return pl.pallas_call(
        _edge_body,
        grid=(grid,),
        in_specs=[
            pl.BlockSpec((BE, D), lambda i: (i, 0)),
            pl.BlockSpec((BE, D), lambda i: (i, 0)),
            pl.BlockSpec((BE, 1), lambda i: (i, 0)),
            pl.BlockSpec((BE, 1), lambda i: (i, 0)),
            pl.BlockSpec((2, D), full),
            pl.BlockSpec((D, D), full),
            pl.BlockSpec((NUM_RBF, D), full),
            pl.BlockSpec((1, D), full),
            pl.BlockSpec((1, D), full),
            pl.BlockSpec((D, D), full),
            pl.BlockSpec((1, D), full),
        ],
        out_specs=pl.BlockSpec((BE, D), lambda i: (i, 0)),
        out_shape=jax.ShapeDtypeStruct((_EH, D), jnp.float32),
    )(ga, gb, d2, etf, emb, w1ct, w1rt, w1d, b1, w2t, b2)


def _node_body(h_ref, s0_ref, s1_ref, cnt_ref, nt_ref, u1at_ref,
               u1bt_ref, c1_ref, u2t_ref, c2_ref, g_ref, bta_ref, out_ref):
    h = h_ref[...]
    s = s0_ref[...] + s1_ref[...]
    cnt = jnp.sum(cnt_ref[0], axis=0, keepdims=True)
    recip = 1.0 / jnp.maximum(cnt, 1.0)
    ii = lax.broadcasted_iota(jnp.int32, (BN, BN), 0)
    jj = lax.broadcasted_iota(jnp.int32, (BN, BN), 1)
    dg = jnp.where(ii == jj, recip, 0.0)
    agg = jnp.dot(dg, s, preferred_element_type=jnp.float32)
    u = _silu(jnp.dot(h, u1at_ref[...], preferred_element_type=jnp.float32)
              + jnp.dot(agg, u1bt_ref[...], preferred_element_type=jnp.float32)
              + c1_ref[...])
    upd = jnp.dot(u, u2t_ref[...], preferred_element_type=jnp.float32) + c2_ref[...]
    pre = h + upd
    mu = jnp.mean(pre, axis=1, keepdims=True)
    cent = pre - mu
    var = jnp.mean(cent * cent, axis=1, keepdims=True)
    ln = cent * lax.rsqrt(var + 1e-05) * g_ref[...] + bta_ref[...]
    out_ref[...] = jnp.where(nt_ref[...] == 1.0, ln, h)


def _node_update(h, s0, s1, cnt, ntf, u1at, u1bt, c1, u2t, c2, g, b):
    grid = N // BN
    full = lambda i: (0, 0)
    return pl.pallas_call(
        _node_body,
        grid=(grid,),
        in_specs=[
            pl.BlockSpec((BN, D), lambda i: (i, 0)),
            pl.BlockSpec((BN, D), lambda i: (i, 0)),
            pl.BlockSpec((BN, D), lambda i: (i, 0)),
            pl.BlockSpec((1, _NW, BN), lambda i: (i, 0, 0)),
            pl.BlockSpec((BN, 1), lambda i: (i, 0)),
            pl.BlockSpec((D, D), full),
            pl.BlockSpec((D, D), full),
            pl.BlockSpec((1, D), full),
            pl.BlockSpec((D, D), full),
            pl.BlockSpec((1, D), full),
            pl.BlockSpec((1, D), full),
            pl.BlockSpec((1, D), full),
        ],
        out_specs=pl.BlockSpec((BN, D), lambda i: (i, 0)),
        out_shape=jax.ShapeDtypeStruct((N, D), jnp.float32),
    )(h, s0, s1, cnt, ntf, u1at, u1bt, c1, u2t, c2, g, b)


def kernel(h, pos, edge_index, edge_type, node_type, emb, W1, b1, W2, b2,
           U1, c1, U2, c2, gamma_ln, beta_ln):
    src32 = edge_index[0].astype(jnp.int32)
    dst32 = edge_index[1].astype(jnp.int32)
    w1at = W1[:, :D].T
    w1bt = W1[:, D:2 * D].T
    w1ct = W1[:, 2 * D:3 * D].T
    w1rt = W1[:, 3 * D:3 * D + NUM_RBF].T
    w1d = W1[:, 3 * D + NUM_RBF][None, :]
    b1r = b1[None, :]
    w2t = W2.T
    b2r = b2[None, :]
    u1at = U1[:, :D].T
    u1bt = U1[:, D:].T
    c1r = c1[None, :]
    u2t = U2.T
    c2r = c2[None, :]
    gr = gamma_ln[None, :]
    br = beta_ln[None, :]
    px = pos[:, 0]
    py = pos[:, 1]
    pz = pos[:, 2]

    ta, tb = _build_tables(h, w1at, w1bt)

    etf = edge_type.astype(jnp.float32)[:, None]

    ga, gb, d2, cntf = _sc_gather(ta, tb, src32, dst32, px, py, pz)
    cnt = cntf.reshape(N // BN, _NW, BN)
    msgp = _edge_mlp(ga, gb, d2[:, None], etf, emb, w1ct, w1rt, w1d, b1r,
                     w2t, b2r)
    zer = jnp.zeros((N, D), jnp.float32)
    dst2 = dst32.reshape(E // _SSUB, _SSUB)
    parts = _sc_scatter(msgp, dst2, zer)

    ntf = node_type.astype(jnp.float32)[:, None]
    return _node_update(h, parts[0], parts[1], cnt, ntf, u1at,
                        u1bt, c1r, u2t, c2r, gr, br)


# speedup vs baseline: 1.0894x; 1.0207x over previous
import functools

import jax
import jax.numpy as jnp
from jax import lax
from jax.experimental import pallas as pl
from jax.experimental.pallas import tpu as pltpu
from jax.experimental.pallas import tpu_sc as plsc

N = 10000
E = 320000
D = 128
NUM_RBF = 32
CUTOFF = 6.0
STEP = CUTOFF / (NUM_RBF - 1)
GAMMA = 1.0 / max(STEP * STEP, 1e-06)

BN = 1000
BE = 2560

_NC = 2
_NS = 16
_NW = _NC * _NS
_L = 16

_NH = 1
_EH = E // _NH
_EPW = _EH // _NW

_GCH = 400
_GSUBS = tuple((o, 80) for o in range(0, 400, 80))
_GNCH = _EPW // _GCH

_SCH = 200
_SSUB = 100
_SNSUB = _SCH // _SSUB
_SNCH = _EPW // _SCH
_RPT = 624
_RTAIL = N - _NS * _RPT


def _silu(x):
    return x * (1.0 / (1.0 + jnp.exp(-x)))


def _table_body(h_ref, w1at_ref, w1bt_ref, a_ref, b_ref):
    h = h_ref[...]
    a_ref[...] = jnp.dot(h, w1at_ref[...], preferred_element_type=jnp.float32)
    b_ref[...] = jnp.dot(h, w1bt_ref[...], preferred_element_type=jnp.float32)


def _build_tables(h, w1at, w1bt):
    grid = N // BN
    return pl.pallas_call(
        _table_body,
        grid=(grid,),
        in_specs=[
            pl.BlockSpec((BN, D), lambda i: (i, 0)),
            pl.BlockSpec((D, D), lambda i: (0, 0)),
            pl.BlockSpec((D, D), lambda i: (0, 0)),
        ],
        out_specs=[
            pl.BlockSpec((BN, D), lambda i: (i, 0)),
            pl.BlockSpec((BN, D), lambda i: (i, 0)),
        ],
        out_shape=[
            jax.ShapeDtypeStruct((N, D), jnp.float32),
            jax.ShapeDtypeStruct((N, D), jnp.float32),
        ],
    )(h, w1at, w1bt)


def _sc_mesh():
    return plsc.VectorSubcoreMesh(core_axis_name="c", subcore_axis_name="s")


def _sc_gather_body(ta, tb, srcr, dstr, pxr, pyr, pzr,
                    ga, gb, d2o, cnto,
                    idxs, idxd, buf, d2b, cntb, px, py, pz, sem):
    cid = lax.axis_index("c")
    sid = lax.axis_index("s")
    wid = sid * _NC + cid
    base = wid * _EPW

    pltpu.sync_copy(pxr, px)
    pltpu.sync_copy(pyr, py)
    pltpu.sync_copy(pzr, pz)
    pltpu.sync_copy(srcr.at[pl.ds(base, _EPW)], idxs)
    pltpu.sync_copy(dstr.at[pl.ds(base, _EPW)], idxd)

    zero16 = jnp.zeros((_L,), jnp.float32)

    def zinit(r, carry):
        cntb[pl.ds(r * _L, _L)] = zero16
        return carry

    lax.fori_loop(0, N // _L, zinit, 0)

    one16 = jnp.ones((_L,), jnp.float32)

    def chunk(i, carry):
        off = base + i * _GCH
        coff = i * _GCH
        ha = []
        for (o, sz) in _GSUBS:
            ha.append(pltpu.async_copy(ta.at[idxs.at[pl.ds(coff + o, sz)]],
                                       buf.at[pl.ds(o, sz)], sem))

        def dcomp(k, c2):
            s16 = pl.ds(coff + k * _L, _L)
            iv_s = idxs[s16]
            iv_d = idxd[s16]
            dx = plsc.load_gather(px, [iv_s]) - plsc.load_gather(px, [iv_d])
            dy = plsc.load_gather(py, [iv_s]) - plsc.load_gather(py, [iv_d])
            dz = plsc.load_gather(pz, [iv_s]) - plsc.load_gather(pz, [iv_d])
            d2b[pl.ds(k * _L, _L)] = dx * dx + dy * dy + dz * dz
            plsc.addupdate_scatter(cntb, [iv_d], one16)
            return c2

        lax.fori_loop(0, _GCH // _L, dcomp, 0)
        pltpu.sync_copy(d2b, d2o.at[pl.ds(off, _GCH)])

        for h in ha:
            h.wait()
        pltpu.sync_copy(buf, ga.at[pl.ds(off, _GCH)])
        hb = []
        for (o, sz) in _GSUBS:
            hb.append(pltpu.async_copy(tb.at[idxd.at[pl.ds(coff + o, sz)]],
                                       buf.at[pl.ds(o, sz)], sem))
        for h in hb:
            h.wait()
        pltpu.sync_copy(buf, gb.at[pl.ds(off, _GCH)])
        return carry

    lax.fori_loop(0, _GNCH, chunk, 0)
    for blk in range(N // BN):
        pltpu.sync_copy(cntb.at[pl.ds(blk * BN, BN)],
                        cnto.at[pl.ds((blk * _NW + wid) * BN, BN)])


def _sc_gather(ta, tb, src, dst, px, py, pz):
    f = pl.kernel(
        _sc_gather_body,
        out_type=[
            jax.ShapeDtypeStruct((_EH, D), jnp.float32),
            jax.ShapeDtypeStruct((_EH, D), jnp.float32),
            jax.ShapeDtypeStruct((_EH,), jnp.float32),
            jax.ShapeDtypeStruct((N * _NW,), jnp.float32),
        ],
        mesh=_sc_mesh(),
        scratch_types=[
            pltpu.VMEM((_EPW,), jnp.int32),
            pltpu.VMEM((_EPW,), jnp.int32),
            pltpu.VMEM((_GCH, D), jnp.float32),
            pltpu.VMEM((_GCH,), jnp.float32),
            pltpu.VMEM((N,), jnp.float32),
            pltpu.VMEM((N,), jnp.float32),
            pltpu.VMEM((N,), jnp.float32),
            pltpu.VMEM((N,), jnp.float32),
            pltpu.SemaphoreType.DMA,
        ],
        compiler_params=pltpu.CompilerParams(needs_layout_passes=False),
    )
    return f(ta, tb, src, dst, px, py, pz)


def _sc_scatter_body(msgp, dst2, zer, out, shared, msgbuf, idxv, sem):
    cid = lax.axis_index("c")
    sid = lax.axis_index("s")
    wid = sid * _NC + cid
    rows = pl.ds(sid * _RPT, _RPT)
    tail = pl.ds(_NS * _RPT, _RTAIL)
    pltpu.sync_copy(zer.at[rows], shared.at[rows])

    @pl.when(sid == 0)
    def _():
        pltpu.sync_copy(zer.at[tail], shared.at[tail])

    plsc.subcore_barrier()
    base = wid * _EPW

    def chunk(i, carry):
        off = base + i * _SCH
        pltpu.sync_copy(msgp.at[pl.ds(off, _SCH)], msgbuf)
        row0 = wid * (_EPW // _SSUB) + i * _SNSUB
        pltpu.sync_copy(dst2.at[pl.ds(row0, _SNSUB)], idxv)
        hs = []
        for j in range(_SNSUB):
            hs.append(pltpu.async_copy(
                msgbuf.at[pl.ds(j * _SSUB, _SSUB)],
                shared.at[idxv.at[j]], sem, add=True))
        for h in hs:
            h.wait()
        return carry

    lax.fori_loop(0, _SNCH, chunk, 0)
    plsc.subcore_barrier()
    pltpu.sync_copy(shared.at[rows], out.at[cid, rows])

    @pl.when(sid == 0)
    def _():
        pltpu.sync_copy(shared.at[tail], out.at[cid, tail])


def _sc_scatter(msgp, dst2, init):
    f = pl.kernel(
        _sc_scatter_body,
        out_type=jax.ShapeDtypeStruct((_NC, N, D), jnp.float32),
        mesh=_sc_mesh(),
        scratch_types=[
            pltpu.MemorySpace.VMEM_SHARED((N, D), jnp.float32),
            pltpu.VMEM((_SCH, D), jnp.float32),
            pltpu.VMEM((_SNSUB, _SSUB), jnp.int32),
            pltpu.SemaphoreType.DMA,
        ],
        compiler_params=pltpu.CompilerParams(needs_layout_passes=False),
    )
    return f(msgp, dst2, init)


def _edge_body(ga_ref, gb_ref, d2_ref, et_ref, emb_ref, w1ct_ref, w1rt_ref,
               w1d_ref, b1_ref, w2t_ref, b2_ref, out_ref):
    gs = ga_ref[...] + gb_ref[...]
    dist = jnp.sqrt(d2_ref[...])
    centers = STEP * lax.broadcasted_iota(jnp.int32, (1, NUM_RBF), 1).astype(jnp.float32)
    diff = dist - centers
    radial = jnp.exp(-GAMMA * diff * diff)
    t = jnp.dot(emb_ref[...], w1ct_ref[...], preferred_element_type=jnp.float32) \
        + b1_ref[...]
    et = et_ref[...]
    tsel = t[0:1, :] * (1.0 - et) + t[1:2, :] * et
    pre1 = (gs + tsel
            + jnp.dot(radial, w1rt_ref[...], preferred_element_type=jnp.float32)
            + dist * w1d_ref[...])
    x = _silu(pre1)
    out_ref[...] = _silu(
        jnp.dot(x, w2t_ref[...], preferred_element_type=jnp.float32)
        + b2_ref[...])


def _edge_mlp(ga, gb, d2, etf, emb, w1ct, w1rt, w1d, b1, w2t, b2):
    grid = _EH // BE
    full = lambda i: (0, 0)
    return pl.pallas_call(
        _edge_body,
        grid=(grid,),
        in_specs=[
            pl.BlockSpec((BE, D), lambda i: (i, 0)),
            pl.BlockSpec((BE, D), lambda i: (i, 0)),
            pl.BlockSpec((BE, 1), lambda i: (i, 0)),
            pl.BlockSpec((BE, 1), lambda i: (i, 0)),
            pl.BlockSpec((2, D), full),
            pl.BlockSpec((D, D), full),
            pl.BlockSpec((NUM_RBF, D), full),
            pl.BlockSpec((1, D), full),
            pl.BlockSpec((1, D), full),
            pl.BlockSpec((D, D), full),
            pl.BlockSpec((1, D), full),
        ],
        out_specs=pl.BlockSpec((BE, D), lambda i: (i, 0)),
        out_shape=jax.ShapeDtypeStruct((_EH, D), jnp.float32),
    )(ga, gb, d2, etf, emb, w1ct, w1rt, w1d, b1, w2t, b2)


def _node_body(h_ref, s0_ref, s1_ref, cnt_ref, nt_ref, u1at_ref,
               u1bt_ref, c1_ref, u2t_ref, c2_ref, g_ref, bta_ref, out_ref):
    h = h_ref[...]
    s = s0_ref[...] + s1_ref[...]
    cnt = jnp.sum(cnt_ref[0], axis=0, keepdims=True)
    recip = 1.0 / jnp.maximum(cnt, 1.0)
    ii = lax.broadcasted_iota(jnp.int32, (BN, BN), 0)
    jj = lax.broadcasted_iota(jnp.int32, (BN, BN), 1)
    dg = jnp.where(ii == jj, recip, 0.0)
    agg = jnp.dot(dg, s, preferred_element_type=jnp.float32)
    u = _silu(jnp.dot(h, u1at_ref[...], preferred_element_type=jnp.float32)
              + jnp.dot(agg, u1bt_ref[...], preferred_element_type=jnp.float32)
              + c1_ref[...])
    upd = jnp.dot(u, u2t_ref[...], preferred_element_type=jnp.float32) + c2_ref[...]
    pre = h + upd
    mu = jnp.mean(pre, axis=1, keepdims=True)
    cent = pre - mu
    var = jnp.mean(cent * cent, axis=1, keepdims=True)
    ln = cent * lax.rsqrt(var + 1e-05) * g_ref[...] + bta_ref[...]
    out_ref[...] = jnp.where(nt_ref[...] == 1.0, ln, h)


def _node_update(h, s0, s1, cnt, ntf, u1at, u1bt, c1, u2t, c2, g, b):
    grid = N // BN
    full = lambda i: (0, 0)
    return pl.pallas_call(
        _node_body,
        grid=(grid,),
        in_specs=[
            pl.BlockSpec((BN, D), lambda i: (i, 0)),
            pl.BlockSpec((BN, D), lambda i: (i, 0)),
            pl.BlockSpec((BN, D), lambda i: (i, 0)),
            pl.BlockSpec((1, _NW, BN), lambda i: (i, 0, 0)),
            pl.BlockSpec((BN, 1), lambda i: (i, 0)),
            pl.BlockSpec((D, D), full),
            pl.BlockSpec((D, D), full),
            pl.BlockSpec((1, D), full),
            pl.BlockSpec((D, D), full),
            pl.BlockSpec((1, D), full),
            pl.BlockSpec((1, D), full),
            pl.BlockSpec((1, D), full),
        ],
        out_specs=pl.BlockSpec((BN, D), lambda i: (i, 0)),
        out_shape=jax.ShapeDtypeStruct((N, D), jnp.float32),
    )(h, s0, s1, cnt, ntf, u1at, u1bt, c1, u2t, c2, g, b)


def kernel(h, pos, edge_index, edge_type, node_type, emb, W1, b1, W2, b2,
           U1, c1, U2, c2, gamma_ln, beta_ln):
    src32 = edge_index[0].astype(jnp.int32)
    dst32 = edge_index[1].astype(jnp.int32)
    w1at = W1[:, :D].T
    w1bt = W1[:, D:2 * D].T
    w1ct = W1[:, 2 * D:3 * D].T
    w1rt = W1[:, 3 * D:3 * D + NUM_RBF].T
    w1d = W1[:, 3 * D + NUM_RBF][None, :]
    b1r = b1[None, :]
    w2t = W2.T
    b2r = b2[None, :]
    u1at = U1[:, :D].T
    u1bt = U1[:, D:].T
    c1r = c1[None, :]
    u2t = U2.T
    c2r = c2[None, :]
    gr = gamma_ln[None, :]
    br = beta_ln[None, :]
    px = pos[:, 0]
    py = pos[:, 1]
    pz = pos[:, 2]

    ta, tb = _build_tables(h, w1at, w1bt)

    etf = edge_type.astype(jnp.float32)[:, None]

    ga, gb, d2, cntf = _sc_gather(ta, tb, src32, dst32, px, py, pz)
    cnt = cntf.reshape(N // BN, _NW, BN)
    msgp = _edge_mlp(ga, gb, d2[:, None], etf, emb, w1ct, w1rt, w1d, b1r,
                     w2t, b2r)
    zer = jnp.zeros((N, D), jnp.float32)
    dst2 = dst32.reshape(E // _SSUB, _SSUB)
    parts = _sc_scatter(msgp, dst2, zer)

    ntf = node_type.astype(jnp.float32)[:, None]
    return _node_update(h, parts[0], parts[1], cnt, ntf, u1at,
                        u1bt, c1r, u2t, c2r, gr, br)
